# baseline (speedup 1.0000x reference)
import jax
import jax.numpy as jnp
from jax.experimental import pallas as pl

_BBC = 512


def _body(x_ref, pos_ref, o_ref):
    nb, nl = x_ref.shape
    d = o_ref.shape[0] // nl
    xt = x_ref[...].T
    y = jnp.broadcast_to(xt[:, None, :], (nl, d, nb)).reshape(nl * d, nb)
    o_ref[...] = y + jnp.broadcast_to(pos_ref[...], (nl * d, nb))


def kernel(x, pos_table):
    B, L = x.shape
    D = pos_table.shape[-1]
    pos_col = pos_table.reshape(L * D, 1)
    y = pl.pallas_call(
        _body,
        grid=(B // _BBC,),
        in_specs=[
            pl.BlockSpec((_BBC, L), lambda i: (i, 0)),
            pl.BlockSpec((L * D, 1), lambda i: (0, 0)),
        ],
        out_specs=pl.BlockSpec((L * D, _BBC), lambda i: (0, i)),
        out_shape=jax.ShapeDtypeStruct((L * D, B), x.dtype),
    )(x, pos_col)
    return y.reshape(L, D, B).transpose(2, 0, 1)


# speedup vs baseline: 1.0178x; 1.0178x over previous
import jax
import jax.numpy as jnp
from jax.experimental import pallas as pl

_BBC = 1024


def _body(x_ref, pos_ref, o_ref):
    nb, nl = x_ref.shape
    d = o_ref.shape[0] // nl
    xt = x_ref[...].T
    y = jnp.broadcast_to(xt[:, None, :], (nl, d, nb)).reshape(nl * d, nb)
    o_ref[...] = y + jnp.broadcast_to(pos_ref[...], (nl * d, nb))


def kernel(x, pos_table):
    B, L = x.shape
    D = pos_table.shape[-1]
    pos_col = pos_table.reshape(L * D, 1)
    y = pl.pallas_call(
        _body,
        grid=(B // _BBC,),
        in_specs=[
            pl.BlockSpec((_BBC, L), lambda i: (i, 0)),
            pl.BlockSpec((L * D, 1), lambda i: (0, 0)),
        ],
        out_specs=pl.BlockSpec((L * D, _BBC), lambda i: (0, i)),
        out_shape=jax.ShapeDtypeStruct((L * D, B), x.dtype),
    )(x, pos_col)
    return y.reshape(L, D, B).transpose(2, 0, 1)
